# baseline (speedup 1.0000x reference)
import jax
import jax.numpy as jnp
from jax.experimental import pallas as pl
from jax.experimental.pallas import tpu as pltpu


def _fused_body(x_ref, embT_hbm, W1T_hbm, b1_hbm, W2T_hbm, b2_hbm,
                out_ref, embT_v, W1T_v, b1_v, W2T_v, b2_v,
                s_emb, s_w1, s_b1, s_w2, s_b2):
    cp_emb = pltpu.make_async_copy(embT_hbm, embT_v, s_emb)
    cp_w1 = pltpu.make_async_copy(W1T_hbm, W1T_v, s_w1)
    cp_b1 = pltpu.make_async_copy(b1_hbm, b1_v, s_b1)
    cp_w2 = pltpu.make_async_copy(W2T_hbm, W2T_v, s_w2)
    cp_b2 = pltpu.make_async_copy(b2_hbm, b2_v, s_b2)
    cp_emb.start()
    cp_w1.start()
    cp_w2.start()
    cp_b1.start()
    cp_b2.start()

    n_vocab = embT_v.shape[1]
    i0 = jnp.clip(x_ref[0], 0, n_vocab - 1)
    i1 = jnp.clip(x_ref[1], 0, n_vocab - 1)
    iota = jax.lax.broadcasted_iota(jnp.int32, (n_vocab, 1), 0)
    oh0 = (iota == i0).astype(jnp.float32)
    oh1 = (iota == i1).astype(jnp.float32)

    cp_emb.wait()
    embT = embT_v[...]
    e0 = jax.lax.dot_general(embT, oh0, (((1,), (0,)), ((), ())),
                             preferred_element_type=jnp.float32)
    e1 = jax.lax.dot_general(embT, oh1, (((1,), (0,)), ((), ())),
                             preferred_element_type=jnp.float32)
    ecat = jnp.concatenate([e0, e1], axis=0)

    cp_w1.wait()
    cp_b1.wait()
    h = jax.lax.dot_general(
        ecat, W1T_v[...], (((0,), (0,)), ((), ())),
        preferred_element_type=jnp.float32,
    ) + jax.lax.reshape(b1_v[...], (1, b1_v.shape[0]))
    h = jnp.maximum(h, 0.0)

    cp_w2.wait()
    cp_b2.wait()
    out_ref[...] = jax.lax.dot_general(
        h, W2T_v[...], (((1,), (0,)), ((), ())),
        preferred_element_type=jnp.float32,
    ) + jax.lax.reshape(b2_v[...], (1, b2_v.shape[0]))


def kernel(x, embed, W1, b1, W2, b2):
    n_vocab, d = embed.shape
    n_hidden = W1.shape[0]
    return pl.pallas_call(
        _fused_body,
        out_shape=jax.ShapeDtypeStruct((1, n_vocab), jnp.float32),
        in_specs=[
            pl.BlockSpec(memory_space=pltpu.SMEM),
            pl.BlockSpec(memory_space=pltpu.MemorySpace.HBM),
            pl.BlockSpec(memory_space=pltpu.MemorySpace.HBM),
            pl.BlockSpec(memory_space=pltpu.MemorySpace.HBM),
            pl.BlockSpec(memory_space=pltpu.MemorySpace.HBM),
            pl.BlockSpec(memory_space=pltpu.MemorySpace.HBM),
        ],
        out_specs=pl.BlockSpec(memory_space=pltpu.VMEM),
        scratch_shapes=[
            pltpu.VMEM((d, n_vocab), jnp.float32),
            pltpu.VMEM((2 * d, n_hidden), jnp.float32),
            pltpu.VMEM((n_hidden,), jnp.float32),
            pltpu.VMEM((n_hidden, n_vocab), jnp.float32),
            pltpu.VMEM((n_vocab,), jnp.float32),
            pltpu.SemaphoreType.DMA,
            pltpu.SemaphoreType.DMA,
            pltpu.SemaphoreType.DMA,
            pltpu.SemaphoreType.DMA,
            pltpu.SemaphoreType.DMA,
        ],
    )(x, embed.T, W1.T, b1, W2.T, b2)


# speedup vs baseline: 1.3382x; 1.3382x over previous
import jax
import jax.numpy as jnp
from jax.experimental import pallas as pl
from jax.experimental.pallas import tpu as pltpu


def _fused_body(x_ref, embT_ref, W1T_ref, b1_ref, W2T_ref, b2_ref,
                out_ref):
    d, n_vocab = embT_ref.shape
    i0 = jnp.clip(x_ref[0], 0, n_vocab - 1)
    i1 = jnp.clip(x_ref[1], 0, n_vocab - 1)
    embT = embT_ref[...]
    lane = jax.lax.broadcasted_iota(jnp.int32, (d, n_vocab), 1)
    zero = jnp.zeros((), jnp.float32)
    e0 = jnp.sum(jnp.where(lane == i0, embT, zero), axis=1,
                 keepdims=True)
    e1 = jnp.sum(jnp.where(lane == i1, embT, zero), axis=1,
                 keepdims=True)
    W1Ta = W1T_ref[:d, :]
    W1Tb = W1T_ref[d:, :]
    h = (
        jax.lax.dot_general(e0, W1Ta, (((0,), (0,)), ((), ())),
                            preferred_element_type=jnp.float32)
        + jax.lax.dot_general(e1, W1Tb, (((0,), (0,)), ((), ())),
                              preferred_element_type=jnp.float32)
        + jax.lax.reshape(b1_ref[...], (1, b1_ref.shape[0]))
    )
    h = jnp.maximum(h, 0.0)
    out_ref[...] = jax.lax.dot_general(
        h, W2T_ref[...], (((1,), (0,)), ((), ())),
        preferred_element_type=jnp.float32,
    ) + jax.lax.reshape(b2_ref[...], (1, b2_ref.shape[0]))


def kernel(x, embed, W1, b1, W2, b2):
    n_vocab = embed.shape[0]
    return pl.pallas_call(
        _fused_body,
        out_shape=jax.ShapeDtypeStruct((1, n_vocab), jnp.float32),
        in_specs=[
            pl.BlockSpec(memory_space=pltpu.SMEM),
            pl.BlockSpec(memory_space=pltpu.VMEM),
            pl.BlockSpec(memory_space=pltpu.VMEM),
            pl.BlockSpec(memory_space=pltpu.VMEM),
            pl.BlockSpec(memory_space=pltpu.VMEM),
            pl.BlockSpec(memory_space=pltpu.VMEM),
        ],
        out_specs=pl.BlockSpec(memory_space=pltpu.VMEM),
    )(x, embed.T, W1.T, b1, W2.T, b2)


# speedup vs baseline: 1.4912x; 1.1143x over previous
import jax
import jax.numpy as jnp
from jax.experimental import pallas as pl
from jax.experimental.pallas import tpu as pltpu


def _fused_body(x_ref, embT_ref, W1T_ref, b1_ref, W2T_ref, b2_ref,
                out_ref):
    d, n_vocab = embT_ref.shape
    i0 = jnp.clip(x_ref[0], 0, n_vocab - 1)
    i1 = jnp.clip(x_ref[1], 0, n_vocab - 1)
    embT = embT_ref[...]
    lane = jax.lax.broadcasted_iota(jnp.int32, (d, n_vocab), 1)
    zero = jnp.zeros((), jnp.float32)
    e0 = jnp.sum(jnp.where(lane == i0, embT, zero), axis=1,
                 keepdims=True)
    e1 = jnp.sum(jnp.where(lane == i1, embT, zero), axis=1,
                 keepdims=True)
    W1Ta = W1T_ref[:d, :]
    W1Tb = W1T_ref[d:, :]
    h = jnp.sum(e0 * W1Ta + e1 * W1Tb, axis=0, keepdims=True) \
        + jax.lax.reshape(b1_ref[...], (1, b1_ref.shape[0]))
    h = jnp.maximum(h, 0.0)
    out_ref[...] = jax.lax.dot_general(
        h, W2T_ref[...], (((1,), (0,)), ((), ())),
        preferred_element_type=jnp.float32,
    ) + jax.lax.reshape(b2_ref[...], (1, b2_ref.shape[0]))


def kernel(x, embed, W1, b1, W2, b2):
    n_vocab = embed.shape[0]
    return pl.pallas_call(
        _fused_body,
        out_shape=jax.ShapeDtypeStruct((1, n_vocab), jnp.float32),
        in_specs=[
            pl.BlockSpec(memory_space=pltpu.SMEM),
            pl.BlockSpec(memory_space=pltpu.VMEM),
            pl.BlockSpec(memory_space=pltpu.VMEM),
            pl.BlockSpec(memory_space=pltpu.VMEM),
            pl.BlockSpec(memory_space=pltpu.VMEM),
            pl.BlockSpec(memory_space=pltpu.VMEM),
        ],
        out_specs=pl.BlockSpec(memory_space=pltpu.VMEM),
    )(x, embed.T, W1.T, b1, W2.T, b2)
